# baseline (speedup 1.0000x reference)
import functools

import jax
import jax.numpy as jnp
from jax import lax
from jax.experimental import pallas as pl
from jax.experimental.pallas import tpu as pltpu
from jax.experimental.pallas import tpu_sc as plsc

N = 10000
D = 128
H = 128
OUT = 128
E = 320000

NC = 2
NS = 16
NW = NC * NS
C = 128
CPW = 80
EPW = CPW * C
EP = NW * EPW
G = 64
ZR = 40
RB = N // ZR
LANES = 16

_GATHER_DNUMS = lax.GatherDimensionNumbers(
    offset_dims=(), collapsed_slice_dims=(0,), start_index_map=(0,))


def _sc_msg_partials(src3, dst3, w3, x):
    mesh = plsc.VectorSubcoreMesh(core_axis_name="c", subcore_axis_name="s")

    @functools.partial(
        pl.kernel,
        mesh=mesh,
        out_type=jax.ShapeDtypeStruct((NC, N, D), jnp.float32),
        scratch_types=[
            pltpu.VMEM((CPW, C), jnp.int32),
            pltpu.VMEM((CPW, C), jnp.int32),
            pltpu.VMEM((CPW, C), jnp.float32),
            pltpu.VMEM((G, D), jnp.float32),
            pltpu.VMEM((G, D), jnp.float32),
            pltpu.VMEM_SHARED((N, D), jnp.float32),
            pltpu.SemaphoreType.DMA,
            pltpu.SemaphoreType.DMA,
            pltpu.SemaphoreType.DMA,
            pltpu.SemaphoreType.DMA,
        ],
    )
    def k(src_hbm, dst_hbm, w_hbm, x_hbm, out_hbm, src_v, dst_v, w_v,
          r0, r1, acc, g0, g1, s0, s1):
        cid = lax.axis_index("c")
        sid = lax.axis_index("s")
        wid = sid * NC + cid

        def scale(rows, kc, cb):
            for j in range(G // LANES):
                w16 = w_v[kc, pl.ds(cb + j * LANES, LANES)]
                for l in range(LANES):
                    wl = lax.gather(
                        w16, jnp.full((LANES, 1), l, jnp.int32),
                        _GATHER_DNUMS, slice_sizes=(1,),
                        mode=lax.GatherScatterMode.PROMISE_IN_BOUNDS)
                    e = j * LANES + l
                    for kk in range(D // LANES):
                        sl = pl.ds(kk * LANES, LANES)
                        rows[e, sl] = rows[e, sl] * wl

        pltpu.sync_copy(src_hbm.at[wid], src_v)
        pltpu.async_copy(x_hbm.at[src_v.at[0, pl.ds(0, G)]], r0, g0)
        pltpu.sync_copy(dst_hbm.at[wid], dst_v)
        pltpu.sync_copy(w_hbm.at[wid], w_v)

        zeros16 = jnp.zeros((LANES,), jnp.float32)
        for e in range(G):
            for kk in range(D // LANES):
                r1[e, pl.ds(kk * LANES, LANES)] = zeros16

        nz = (RB + NS - 1) // NS

        def zacc(j, carry):
            ch = sid + j * NS

            @pl.when(ch < RB)
            def _():
                pltpu.sync_copy(r1.at[pl.ds(0, ZR)], acc.at[pl.ds(ch * ZR, ZR)])

            return carry

        lax.fori_loop(0, nz, zacc, 0)
        plsc.subcore_barrier()

        def pair(kc, carry):
            for b in range(2):
                cb = b * G
                rb, ro = (r0, r1) if b == 0 else (r1, r0)
                gb, go = (g0, g1) if b == 0 else (g1, g0)
                sb, so = (s0, s1) if b == 0 else (s1, s0)
                idx = src_v.at[kc, pl.ds(cb, G)]
                didx = dst_v.at[kc, pl.ds(cb, G)]
                if b == 0:
                    nidx = src_v.at[kc, pl.ds(G, G)]
                else:
                    nidx = src_v.at[kc + 1, pl.ds(0, G)]
                pltpu.make_async_copy(x_hbm.at[idx], rb, gb).wait()

                @pl.when((kc > 0) | (b > 0))
                def _():
                    pltpu.make_async_copy(ro, acc.at[didx], so).wait()

                pltpu.async_copy(x_hbm.at[nidx], ro, go)
                scale(rb, kc, cb)
                pltpu.async_copy(rb, acc.at[didx], sb, add=True)
            return carry

        lax.fori_loop(0, CPW - 1, pair, 0)

        kc = CPW - 1
        pltpu.make_async_copy(x_hbm.at[src_v.at[kc, pl.ds(0, G)]], r0, g0).wait()
        pltpu.make_async_copy(r1, acc.at[dst_v.at[kc - 1, pl.ds(G, G)]],
                              s1).wait()
        pltpu.async_copy(x_hbm.at[src_v.at[kc, pl.ds(G, G)]], r1, g1)
        scale(r0, kc, 0)
        pltpu.async_copy(r0, acc.at[dst_v.at[kc, pl.ds(0, G)]], s0, add=True)
        pltpu.make_async_copy(x_hbm.at[src_v.at[kc, pl.ds(G, G)]], r1, g1).wait()
        scale(r1, kc, G)
        pltpu.make_async_copy(r0, acc.at[dst_v.at[kc, pl.ds(0, G)]], s0).wait()
        pltpu.sync_copy(r1, acc.at[dst_v.at[kc, pl.ds(G, G)]], add=True)
        plsc.subcore_barrier()

        def wout(j, carry):
            ch = sid + j * NS

            @pl.when(ch < RB)
            def _():
                pltpu.sync_copy(acc.at[pl.ds(ch * ZR, ZR)],
                                out_hbm.at[cid, pl.ds(ch * ZR, ZR)])

            return carry

        lax.fori_loop(0, nz, wout, 0)

    return k(src3, dst3, w3, x)


def _dense_body(x_ref, m0_ref, m1_ref, win_ref, bin_ref, wout_ref, bout_ref,
                o_ref):
    xb = x_ref[...]
    mb = m0_ref[...] + m1_ref[...]
    win = win_ref[...]
    h = (jnp.dot(xb, win[:, :D].T, preferred_element_type=jnp.float32)
         + jnp.dot(mb, win[:, D:].T, preferred_element_type=jnp.float32)
         + bin_ref[...])
    h = jnp.where(h > 0, h, jnp.exp(jnp.minimum(h, 0.0)) - 1.0)
    o_ref[...] = (jnp.dot(h, wout_ref[...].T,
                          preferred_element_type=jnp.float32) + bout_ref[...])


def _tc_dense(x, m0, m1, W_in, b_in, W_out, b_out):
    BN = 1000
    grid = (N // BN,)
    return pl.pallas_call(
        _dense_body,
        grid=grid,
        in_specs=[
            pl.BlockSpec((BN, D), lambda i: (i, 0)),
            pl.BlockSpec((BN, D), lambda i: (i, 0)),
            pl.BlockSpec((BN, D), lambda i: (i, 0)),
            pl.BlockSpec((H, 2 * D), lambda i: (0, 0)),
            pl.BlockSpec((1, H), lambda i: (0, 0)),
            pl.BlockSpec((OUT, H), lambda i: (0, 0)),
            pl.BlockSpec((1, OUT), lambda i: (0, 0)),
        ],
        out_specs=pl.BlockSpec((BN, OUT), lambda i: (i, 0)),
        out_shape=jax.ShapeDtypeStruct((N, OUT), jnp.float32),
    )(x, m0, m1, W_in, b_in.reshape(1, H), W_out, b_out.reshape(1, OUT))


def kernel(feature_data, edge_info, edge_weights, W_in, b_in, W_out, b_out):
    pad = EP - E
    src3 = jnp.pad(edge_info[0], (0, pad)).reshape(NW, CPW, C)
    dst3 = jnp.concatenate(
        [edge_info[1], jnp.arange(pad, dtype=jnp.int32) % N]
    ).reshape(NW, CPW, C)
    w3 = jnp.pad(edge_weights, (0, pad)).reshape(NW, CPW, C)
    msgp = _sc_msg_partials(src3, dst3, w3, feature_data)
    return _tc_dense(feature_data, msgp[0], msgp[1], W_in, b_in, W_out, b_out)


# speedup vs baseline: 2.0459x; 2.0459x over previous
import functools

import jax
import jax.numpy as jnp
from jax import lax
from jax.experimental import pallas as pl
from jax.experimental.pallas import tpu as pltpu
from jax.experimental.pallas import tpu_sc as plsc

N = 10000
D = 128
H = 128
OUT = 128
E = 320000

NC = 2
NS = 16
NW = NC * NS
C = 128
CPW = 80
EPW = CPW * C
EP = NW * EPW
G = 64
ZR = 40
RB = N // ZR
LANES = 16

_GATHER_DNUMS = lax.GatherDimensionNumbers(
    offset_dims=(), collapsed_slice_dims=(0,), start_index_map=(0,))


def _sc_msg_partials(src3, dst3, w3, x):
    mesh = plsc.VectorSubcoreMesh(core_axis_name="c", subcore_axis_name="s")

    @functools.partial(
        pl.kernel,
        mesh=mesh,
        out_type=jax.ShapeDtypeStruct((NC, N, D), jnp.float32),
        scratch_types=[
            pltpu.VMEM((CPW, C), jnp.int32),
            pltpu.VMEM((CPW, C), jnp.int32),
            pltpu.VMEM((CPW, C), jnp.float32),
            pltpu.VMEM((G, D), jnp.float32),
            pltpu.VMEM((G, D), jnp.float32),
            pltpu.VMEM_SHARED((N, D), jnp.float32),
            pltpu.SemaphoreType.DMA,
            pltpu.SemaphoreType.DMA,
            pltpu.SemaphoreType.DMA,
            pltpu.SemaphoreType.DMA,
        ],
    )
    def k(src_hbm, dst_hbm, w_hbm, x_hbm, out_hbm, src_v, dst_v, w_v,
          r0, r1, acc, g0, g1, s0, s1):
        cid = lax.axis_index("c")
        sid = lax.axis_index("s")
        wid = sid * NC + cid

        def scale(rows, kc, cb):
            for j in range(G // LANES):
                w16 = w_v[kc, pl.ds(cb + j * LANES, LANES)]
                for l in range(LANES):
                    wl = lax.gather(
                        w16, jnp.full((LANES, 1), l, jnp.int32),
                        _GATHER_DNUMS, slice_sizes=(1,),
                        mode=lax.GatherScatterMode.PROMISE_IN_BOUNDS)
                    e = j * LANES + l
                    for kk in range(D // LANES):
                        sl = pl.ds(kk * LANES, LANES)
                        rows[e, sl] = rows[e, sl] * wl

        pltpu.sync_copy(src_hbm.at[wid], src_v)
        pltpu.async_copy(x_hbm.at[src_v.at[0, pl.ds(0, G)]], r0, g0)
        pltpu.sync_copy(dst_hbm.at[wid], dst_v)
        pltpu.sync_copy(w_hbm.at[wid], w_v)

        zeros16 = jnp.zeros((LANES,), jnp.float32)
        for e in range(G):
            for kk in range(D // LANES):
                r1[e, pl.ds(kk * LANES, LANES)] = zeros16

        nz = (RB + NS - 1) // NS

        def zacc(j, carry):
            ch = sid + j * NS

            @pl.when(ch < RB)
            def _():
                pltpu.sync_copy(r1.at[pl.ds(0, ZR)], acc.at[pl.ds(ch * ZR, ZR)])

            return carry

        lax.fori_loop(0, nz, zacc, 0)
        plsc.subcore_barrier()

        def pair(kc, carry):
            for b in range(2):
                cb = b * G
                rb, ro = (r0, r1) if b == 0 else (r1, r0)
                gb, go = (g0, g1) if b == 0 else (g1, g0)
                sb, so = (s0, s1) if b == 0 else (s1, s0)
                idx = src_v.at[kc, pl.ds(cb, G)]
                didx = dst_v.at[kc, pl.ds(cb, G)]
                if b == 0:
                    nidx = src_v.at[kc, pl.ds(G, G)]
                else:
                    nidx = src_v.at[kc + 1, pl.ds(0, G)]
                pltpu.make_async_copy(x_hbm.at[idx], rb, gb).wait()

                @pl.when((kc > 0) | (b > 0))
                def _():
                    pltpu.make_async_copy(ro, acc.at[didx], so).wait()

                pltpu.async_copy(x_hbm.at[nidx], ro, go)
                scale(rb, kc, cb)
                pltpu.async_copy(rb, acc.at[didx], sb, add=True)
            return carry

        lax.fori_loop(0, CPW - 1, pair, 0)

        kc = CPW - 1
        pltpu.make_async_copy(x_hbm.at[src_v.at[kc, pl.ds(0, G)]], r0, g0).wait()
        pltpu.make_async_copy(r1, acc.at[dst_v.at[kc - 1, pl.ds(G, G)]],
                              s1).wait()
        pltpu.async_copy(x_hbm.at[src_v.at[kc, pl.ds(G, G)]], r1, g1)
        scale(r0, kc, 0)
        pltpu.async_copy(r0, acc.at[dst_v.at[kc, pl.ds(0, G)]], s0, add=True)
        pltpu.make_async_copy(x_hbm.at[src_v.at[kc, pl.ds(G, G)]], r1, g1).wait()
        scale(r1, kc, G)
        pltpu.make_async_copy(r0, acc.at[dst_v.at[kc, pl.ds(0, G)]], s0).wait()
        pltpu.sync_copy(r1, acc.at[dst_v.at[kc, pl.ds(G, G)]], add=True)
        plsc.subcore_barrier()

        def wout(j, carry):
            ch = sid + j * NS

            @pl.when(ch < RB)
            def _():
                pltpu.sync_copy(acc.at[pl.ds(ch * ZR, ZR)],
                                out_hbm.at[cid, pl.ds(ch * ZR, ZR)])

            return carry

        lax.fori_loop(0, nz, wout, 0)

    return k(src3, dst3, w3, x)


def _dense_body(x_ref, m0_ref, m1_ref, win_ref, bin_ref, wout_ref, bout_ref,
                o_ref):
    xb = x_ref[...]
    mb = m0_ref[...] + m1_ref[...]
    win = win_ref[...]
    h = (jnp.dot(xb, win[:, :D].T, preferred_element_type=jnp.float32)
         + jnp.dot(mb, win[:, D:].T, preferred_element_type=jnp.float32)
         + bin_ref[...])
    h = jnp.where(h > 0, h, jnp.exp(jnp.minimum(h, 0.0)) - 1.0)
    o_ref[...] = (jnp.dot(h, wout_ref[...].T,
                          preferred_element_type=jnp.float32) + bout_ref[...])


def _tc_dense(x, m0, m1, W_in, b_in, W_out, b_out):
    BN = 1000
    grid = (N // BN,)
    return pl.pallas_call(
        _dense_body,
        grid=grid,
        in_specs=[
            pl.BlockSpec((BN, D), lambda i: (i, 0)),
            pl.BlockSpec((BN, D), lambda i: (i, 0)),
            pl.BlockSpec((BN, D), lambda i: (i, 0)),
            pl.BlockSpec((H, 2 * D), lambda i: (0, 0)),
            pl.BlockSpec((1, H), lambda i: (0, 0)),
            pl.BlockSpec((OUT, H), lambda i: (0, 0)),
            pl.BlockSpec((1, OUT), lambda i: (0, 0)),
        ],
        out_specs=pl.BlockSpec((BN, OUT), lambda i: (i, 0)),
        out_shape=jax.ShapeDtypeStruct((N, OUT), jnp.float32),
    )(x, m0, m1, W_in, b_in.reshape(1, H), W_out, b_out.reshape(1, OUT))


def kernel(feature_data, edge_info, edge_weights, W_in, b_in, W_out, b_out):
    pad = EP - E
    ppw = pad // NW
    spread = (jnp.arange(pad, dtype=jnp.int32) % N).reshape(NW, ppw)

    def lay(real, padv):
        return jnp.concatenate(
            [real.reshape(NW, E // NW), padv], axis=1).reshape(NW, CPW, C)

    src3 = lay(edge_info[0], spread)
    dst3 = lay(edge_info[1], spread)
    w3 = lay(edge_weights, jnp.zeros((NW, ppw), jnp.float32))
    msgp = _sc_msg_partials(src3, dst3, w3, feature_data)
    return _tc_dense(feature_data, msgp[0], msgp[1], W_in, b_in, W_out, b_out)


# speedup vs baseline: 2.0539x; 1.0039x over previous
import functools

import jax
import jax.numpy as jnp
from jax import lax
from jax.experimental import pallas as pl
from jax.experimental.pallas import tpu as pltpu
from jax.experimental.pallas import tpu_sc as plsc

N = 10000
D = 128
H = 128
OUT = 128
E = 320000

NC = 2
NS = 16
NW = NC * NS
C = 128
CPW = 80
EPW = CPW * C
EP = NW * EPW
G = 64
ZR = 40
RB = N // ZR
LANES = 16

_GATHER_DNUMS = lax.GatherDimensionNumbers(
    offset_dims=(), collapsed_slice_dims=(0,), start_index_map=(0,))


def _sc_msg_partials(src3, dst3, w3, x):
    mesh = plsc.VectorSubcoreMesh(core_axis_name="c", subcore_axis_name="s")

    @functools.partial(
        pl.kernel,
        mesh=mesh,
        out_type=jax.ShapeDtypeStruct((NC, N, D), jnp.float32),
        scratch_types=[
            pltpu.VMEM((CPW, C), jnp.int32),
            pltpu.VMEM((CPW, C), jnp.int32),
            pltpu.VMEM((CPW, C), jnp.float32),
            pltpu.VMEM((G, D), jnp.float32),
            pltpu.VMEM((G, D), jnp.float32),
            pltpu.VMEM_SHARED((N, D), jnp.float32),
            pltpu.SemaphoreType.DMA,
            pltpu.SemaphoreType.DMA,
            pltpu.SemaphoreType.DMA,
            pltpu.SemaphoreType.DMA,
        ],
    )
    def k(src_hbm, dst_hbm, w_hbm, x_hbm, out_hbm, src_v, dst_v, w_v,
          r0, r1, acc, g0, g1, s0, s1):
        cid = lax.axis_index("c")
        sid = lax.axis_index("s")
        wid = sid * NC + cid

        def scale(rows, kc, cb):
            for j in range(G // LANES):
                w16 = w_v[kc, pl.ds(cb + j * LANES, LANES)]
                for l in range(LANES):
                    wl = lax.gather(
                        w16, jnp.full((LANES, 1), l, jnp.int32),
                        _GATHER_DNUMS, slice_sizes=(1,),
                        mode=lax.GatherScatterMode.PROMISE_IN_BOUNDS)
                    e = j * LANES + l
                    for kk in range(D // LANES):
                        sl = pl.ds(kk * LANES, LANES)
                        rows[e, sl] = rows[e, sl] * wl

        pltpu.sync_copy(src_hbm.at[wid], src_v)
        pltpu.async_copy(x_hbm.at[src_v.at[0, pl.ds(0, G)]], r0, g0)
        pltpu.sync_copy(dst_hbm.at[wid], dst_v)
        pltpu.sync_copy(w_hbm.at[wid], w_v)

        zeros16 = jnp.zeros((LANES,), jnp.float32)
        for e in range(G):
            for kk in range(D // LANES):
                r1[e, pl.ds(kk * LANES, LANES)] = zeros16

        nz = (RB + NS - 1) // NS

        def zacc(j, carry):
            ch = sid + j * NS

            @pl.when(ch < RB)
            def _():
                pltpu.sync_copy(r1.at[pl.ds(0, ZR)], acc.at[pl.ds(ch * ZR, ZR)])

            return carry

        lax.fori_loop(0, nz, zacc, 0)
        plsc.subcore_barrier()

        def pair(kc, carry):
            for b in range(2):
                cb = b * G
                rb, ro = (r0, r1) if b == 0 else (r1, r0)
                gb, go = (g0, g1) if b == 0 else (g1, g0)
                sb, so = (s0, s1) if b == 0 else (s1, s0)
                idx = src_v.at[kc, pl.ds(cb, G)]
                didx = dst_v.at[kc, pl.ds(cb, G)]
                if b == 0:
                    nidx = src_v.at[kc, pl.ds(G, G)]
                else:
                    nidx = src_v.at[kc + 1, pl.ds(0, G)]
                pltpu.make_async_copy(x_hbm.at[idx], rb, gb).wait()

                @pl.when((kc > 0) | (b > 0))
                def _():
                    pltpu.make_async_copy(ro, acc.at[didx], so).wait()

                pltpu.async_copy(x_hbm.at[nidx], ro, go)
                scale(rb, kc, cb)
                pltpu.async_copy(rb, acc.at[didx], sb, add=True)
            return carry

        lax.fori_loop(0, CPW - 1, pair, 0)

        kc = CPW - 1
        pltpu.make_async_copy(x_hbm.at[src_v.at[kc, pl.ds(0, G)]], r0, g0).wait()
        pltpu.make_async_copy(r1, acc.at[dst_v.at[kc - 1, pl.ds(G, G)]],
                              s1).wait()
        pltpu.async_copy(x_hbm.at[src_v.at[kc, pl.ds(G, G)]], r1, g1)
        scale(r0, kc, 0)
        pltpu.async_copy(r0, acc.at[dst_v.at[kc, pl.ds(0, G)]], s0, add=True)
        pltpu.make_async_copy(x_hbm.at[src_v.at[kc, pl.ds(G, G)]], r1, g1).wait()
        scale(r1, kc, G)
        pltpu.make_async_copy(r0, acc.at[dst_v.at[kc, pl.ds(0, G)]], s0).wait()
        pltpu.sync_copy(r1, acc.at[dst_v.at[kc, pl.ds(G, G)]], add=True)
        plsc.subcore_barrier()

        def wout(j, carry):
            ch = sid + j * NS

            @pl.when(ch < RB)
            def _():
                pltpu.sync_copy(acc.at[pl.ds(ch * ZR, ZR)],
                                out_hbm.at[cid, pl.ds(ch * ZR, ZR)])

            return carry

        lax.fori_loop(0, nz, wout, 0)

    return k(src3, dst3, w3, x)


BN = 1000


def _xw_body(x_ref, wina_ref, bin_ref, o_ref):
    o_ref[...] = (jnp.dot(x_ref[...], wina_ref[...].T,
                          preferred_element_type=jnp.float32) + bin_ref[...])


def _tc_xw(x, W_in_a, b_in):
    return pl.pallas_call(
        _xw_body,
        grid=(N // BN,),
        in_specs=[
            pl.BlockSpec((BN, D), lambda i: (i, 0)),
            pl.BlockSpec((H, D), lambda i: (0, 0)),
            pl.BlockSpec((1, H), lambda i: (0, 0)),
        ],
        out_specs=pl.BlockSpec((BN, H), lambda i: (i, 0)),
        out_shape=jax.ShapeDtypeStruct((N, H), jnp.float32),
    )(x, W_in_a, b_in.reshape(1, H))


def _dense_body(xw_ref, m0_ref, m1_ref, winb_ref, wout_ref, bout_ref, o_ref):
    mb = m0_ref[...] + m1_ref[...]
    h = xw_ref[...] + jnp.dot(mb, winb_ref[...].T,
                              preferred_element_type=jnp.float32)
    h = jnp.where(h > 0, h, jnp.exp(jnp.minimum(h, 0.0)) - 1.0)
    o_ref[...] = (jnp.dot(h, wout_ref[...].T,
                          preferred_element_type=jnp.float32) + bout_ref[...])


def _tc_dense(xw, m0, m1, W_in_b, W_out, b_out):
    return pl.pallas_call(
        _dense_body,
        grid=(N // BN,),
        in_specs=[
            pl.BlockSpec((BN, H), lambda i: (i, 0)),
            pl.BlockSpec((BN, D), lambda i: (i, 0)),
            pl.BlockSpec((BN, D), lambda i: (i, 0)),
            pl.BlockSpec((H, D), lambda i: (0, 0)),
            pl.BlockSpec((OUT, H), lambda i: (0, 0)),
            pl.BlockSpec((1, OUT), lambda i: (0, 0)),
        ],
        out_specs=pl.BlockSpec((BN, OUT), lambda i: (i, 0)),
        out_shape=jax.ShapeDtypeStruct((N, OUT), jnp.float32),
    )(xw, m0, m1, W_in_b, W_out, b_out.reshape(1, OUT))


def kernel(feature_data, edge_info, edge_weights, W_in, b_in, W_out, b_out):
    pad = EP - E
    ppw = pad // NW
    spread = (jnp.arange(pad, dtype=jnp.int32) % N).reshape(NW, ppw)

    def lay(real, padv):
        return jnp.concatenate(
            [real.reshape(NW, E // NW), padv], axis=1).reshape(NW, CPW, C)

    src3 = lay(edge_info[0], spread)
    dst3 = lay(edge_info[1], spread)
    w3 = lay(edge_weights, jnp.zeros((NW, ppw), jnp.float32))
    xw = _tc_xw(feature_data, W_in[:, :D], b_in)
    msgp = _sc_msg_partials(src3, dst3, w3, feature_data)
    return _tc_dense(xw, msgp[0], msgp[1], W_in[:, D:], W_out, b_out)
